# Initial kernel scaffold; baseline (speedup 1.0000x reference)
import jax
import jax.numpy as jnp
from jax.experimental import pallas as pl


def kernel(query, support):
    raise NotImplementedError("write your pallas kernel here")


# speedup vs baseline: 14.0118x; 14.0118x over previous
import jax
import jax.numpy as jnp
from jax.experimental import pallas as pl

K_NB = 16
MT = 128


def _knn_block(q_ref, s_ref, qq_ref, ss_ref, vals_ref, idx_ref):
    q = q_ref[0]
    s = s_ref[0]
    n = s.shape[0]

    cross = jax.lax.dot_general(
        q, s, (((1,), (1,)), ((), ())), preferred_element_type=jnp.float32)
    qq = qq_ref[0]
    ss = ss_ref[0]
    d2 = (qq + ss) - 2.0 * cross
    dist = jnp.sqrt(jnp.maximum(d2, 0.0))

    lane = jax.lax.broadcasted_iota(jnp.int32, dist.shape, 1)
    val_cols = []
    idx_cols = []
    for _ in range(K_NB):
        m = jnp.min(dist, axis=1, keepdims=True)
        am = jnp.min(jnp.where(dist <= m, lane, n), axis=1, keepdims=True)
        val_cols.append(m)
        idx_cols.append(am)
        dist = jnp.where(lane == am, jnp.inf, dist)

    vals_ref[0] = jnp.concatenate(val_cols, axis=1)
    idx_ref[0] = jnp.concatenate(idx_cols, axis=1)


def kernel(query, support):
    b, m, c = query.shape
    _, n, _ = support.shape
    qq = jnp.sum(query * query, axis=-1, keepdims=True)
    ss = jnp.sum(support * support, axis=-1)[:, None, :]
    grid = (b, m // MT)
    vals, idx = pl.pallas_call(
        _knn_block,
        grid=grid,
        in_specs=[
            pl.BlockSpec((1, MT, c), lambda bi, mi: (bi, mi, 0)),
            pl.BlockSpec((1, n, c), lambda bi, mi: (bi, 0, 0)),
            pl.BlockSpec((1, MT, 1), lambda bi, mi: (bi, mi, 0)),
            pl.BlockSpec((1, 1, n), lambda bi, mi: (bi, 0, 0)),
        ],
        out_specs=[
            pl.BlockSpec((1, MT, K_NB), lambda bi, mi: (bi, mi, 0)),
            pl.BlockSpec((1, MT, K_NB), lambda bi, mi: (bi, mi, 0)),
        ],
        out_shape=[
            jax.ShapeDtypeStruct((b, m, K_NB), jnp.float32),
            jax.ShapeDtypeStruct((b, m, K_NB), jnp.int32),
        ],
    )(query, support, qq, ss)
    return (vals, idx)
